# Initial kernel scaffold; baseline (speedup 1.0000x reference)
import jax
import jax.numpy as jnp
from jax.experimental import pallas as pl


def kernel(x, edge_index, W0, b0, W1, b1, W2, b2, W3, b3, W4, b4):
    raise NotImplementedError("write your pallas kernel here")


# speedup vs baseline: 8.4659x; 8.4659x over previous
import functools

import jax
import jax.numpy as jnp
from jax import lax
from jax.experimental import pallas as pl
from jax.experimental.pallas import tpu as pltpu
from jax.experimental.pallas import tpu_sc as plsc

NN = 10000
NP = 10240
E = 640000
NC, NS = 2, 16
NW = NC * NS
EW = E // NW
K = 80
NB = EW // K
SL = NP // NS
WF = 128
F32 = jnp.float32


def _make_edge_agg(width):
    if width == 1:
        h_t = jax.ShapeDtypeStruct((NP,), F32)
        out_t = jax.ShapeDtypeStruct((NC, NP), F32)
        rows_t = pltpu.VMEM((K,), F32)
        acc_t = pltpu.VMEM_SHARED((NP,), F32)
    else:
        h_t = jax.ShapeDtypeStruct((NP, width), F32)
        out_t = jax.ShapeDtypeStruct((NC, NP, width), F32)
        rows_t = pltpu.VMEM((K, width), F32)
        acc_t = pltpu.VMEM_SHARED((NP, width), F32)

    mesh = plsc.VectorSubcoreMesh(
        core_axis_name="c", subcore_axis_name="s", num_cores=NC, num_subcores=NS
    )

    @functools.partial(
        pl.kernel,
        out_type=out_t,
        mesh=mesh,
        scratch_types=[
            pltpu.VMEM((K,), jnp.int32),
            pltpu.VMEM((K,), jnp.int32),
            rows_t,
            acc_t,
            pltpu.SemaphoreType.DMA,
        ],
    )
    def agg(h_hbm, src_hbm, dst_hbm, zero_hbm, out_hbm, src_v, dst_v, rows_v, acc, sem):
        c = lax.axis_index("c")
        s = lax.axis_index("s")
        if width == 1:
            pltpu.sync_copy(zero_hbm.at[pl.ds(s * SL, SL)], acc.at[pl.ds(s * SL, SL)])
        else:
            pltpu.sync_copy(zero_hbm.at[pl.ds(s * SL, SL), :], acc.at[pl.ds(s * SL, SL), :])
        plsc.subcore_barrier()

        base = (c * NS + s) * EW

        def step(i, carry):
            off = base + i * K
            pltpu.sync_copy(src_hbm.at[pl.ds(off, K)], src_v)
            pltpu.sync_copy(dst_hbm.at[pl.ds(off, K)], dst_v)
            pltpu.async_copy(h_hbm.at[src_v], rows_v, sem).wait()
            pltpu.sync_copy(rows_v, acc.at[dst_v], add=True)
            return carry

        lax.fori_loop(0, NB, step, 0)
        plsc.subcore_barrier()
        if width == 1:
            pltpu.sync_copy(acc.at[pl.ds(s * SL, SL)], out_hbm.at[c, pl.ds(s * SL, SL)])
        else:
            pltpu.sync_copy(
                acc.at[pl.ds(s * SL, SL), :], out_hbm.at[c, pl.ds(s * SL, SL), :]
            )

    def call(h, src, dst):
        zero = jnp.zeros(h_t.shape, F32)
        return agg(h, src, dst, zero)

    return call


_agg1 = _make_edge_agg(1)
_aggF = _make_edge_agg(WF)


_R = 2048


_NR = NP // 128


def _prep_body(u_ref, y_ref, dinv_ref, hs_ref):
    indeg = u_ref[0] + u_ref[1]
    dinv = lax.rsqrt(indeg + 1.0)
    dinv_ref[...] = dinv
    hs_ref[...] = dinv * y_ref[...]


def _prep(u, y):
    return pl.pallas_call(
        _prep_body,
        out_shape=(
            jax.ShapeDtypeStruct((_NR, 128), F32),
            jax.ShapeDtypeStruct((_NR, 128), F32),
        ),
    )(u, y)


def _l0_body(u_ref, hs_ref, dinv_ref, w_ref, b_ref, o_ref):
    dinv = dinv_ref[...]
    agg = dinv * (u_ref[0] + u_ref[1] + hs_ref[...])
    h = jnp.tanh(agg * w_ref[...] + b_ref[...])
    o_ref[...] = dinv * h


def _l0(u, hs, dinv, w0, b0):
    g = NP // _R
    return pl.pallas_call(
        _l0_body,
        grid=(g,),
        in_specs=[
            pl.BlockSpec((2, _R, 1), lambda i: (0, i, 0)),
            pl.BlockSpec((_R, 1), lambda i: (i, 0)),
            pl.BlockSpec((_R, 1), lambda i: (i, 0)),
            pl.BlockSpec((1, WF), lambda i: (0, 0)),
            pl.BlockSpec((1, WF), lambda i: (0, 0)),
        ],
        out_specs=pl.BlockSpec((_R, WF), lambda i: (i, 0)),
        out_shape=jax.ShapeDtypeStruct((NP, WF), F32),
    )(u, hs, dinv, w0, b0)


def _lmid_body(u_ref, hs_ref, dinv_ref, w_ref, b_ref, o_ref):
    dinv = dinv_ref[...]
    agg = dinv * (u_ref[0] + u_ref[1] + hs_ref[...])
    h = jnp.tanh(
        jnp.dot(agg, w_ref[...], preferred_element_type=F32) + b_ref[...]
    )
    o_ref[...] = dinv * h


def _lmid(u, hs, dinv, w, b):
    g = NP // _R
    return pl.pallas_call(
        _lmid_body,
        grid=(g,),
        in_specs=[
            pl.BlockSpec((2, _R, WF), lambda i: (0, i, 0)),
            pl.BlockSpec((_R, WF), lambda i: (i, 0)),
            pl.BlockSpec((_R, 1), lambda i: (i, 0)),
            pl.BlockSpec((WF, WF), lambda i: (0, 0)),
            pl.BlockSpec((1, WF), lambda i: (0, 0)),
        ],
        out_specs=pl.BlockSpec((_R, WF), lambda i: (i, 0)),
        out_shape=jax.ShapeDtypeStruct((NP, WF), F32),
    )(u, hs, dinv, w, b)


def _l3z_body(u_ref, hs_ref, dinv_ref, w3_ref, b3_ref, w4_ref, o_ref):
    dinv = dinv_ref[...]
    agg = dinv * (u_ref[0] + u_ref[1] + hs_ref[...])
    h4 = jnp.tanh(
        jnp.dot(agg, w3_ref[...], preferred_element_type=F32) + b3_ref[...]
    )
    z = jnp.dot(h4, w4_ref[...], preferred_element_type=F32)
    o_ref[...] = dinv * z


def _l3z(u, hs, dinv, w3, b3, w4):
    g = NP // _R
    return pl.pallas_call(
        _l3z_body,
        grid=(g,),
        in_specs=[
            pl.BlockSpec((2, _R, WF), lambda i: (0, i, 0)),
            pl.BlockSpec((_R, WF), lambda i: (i, 0)),
            pl.BlockSpec((_R, 1), lambda i: (i, 0)),
            pl.BlockSpec((WF, WF), lambda i: (0, 0)),
            pl.BlockSpec((1, WF), lambda i: (0, 0)),
            pl.BlockSpec((WF, 1), lambda i: (0, 0)),
        ],
        out_specs=pl.BlockSpec((_R, 1), lambda i: (i, 0)),
        out_shape=jax.ShapeDtypeStruct((NP, 1), F32),
    )(u, hs, dinv, w3, b3, w4)


def _fin_body(u_ref, zs_ref, dinv_ref, b4_ref, y_ref, ka_ref, kb_ref, kc_ref,
              coef_ref, k_ref, ya_ref, hsa_ref):
    dinv = dinv_ref[...]
    k = dinv * (u_ref[0] + u_ref[1] + zs_ref[...]) + b4_ref[0, 0]
    y_arg = (
        y_ref[...]
        + coef_ref[0, 0] * ka_ref[...]
        + coef_ref[0, 1] * kb_ref[...]
        + coef_ref[0, 2] * kc_ref[...]
        + coef_ref[0, 3] * k
    )
    k_ref[...] = k
    ya_ref[...] = y_arg
    hsa_ref[...] = dinv * y_arg


def _fin(u, zs, dinv, b4, y, ka, kb, kc, coef):
    return pl.pallas_call(
        _fin_body,
        out_shape=(
            jax.ShapeDtypeStruct((_NR, 128), F32),
            jax.ShapeDtypeStruct((_NR, 128), F32),
            jax.ShapeDtypeStruct((_NR, 128), F32),
        ),
    )(u, zs, dinv, b4, y, ka, kb, kc, coef)


def _padw(w, rows, cols):
    return jnp.pad(w, ((0, rows - w.shape[0]), (0, cols - w.shape[1])))


def kernel(x, edge_index, W0, b0, W1, b1, W2, b2, W3, b3, W4, b4):
    bsz, nn, _ = x.shape
    n = bsz * nn
    pad = NP - n
    y0 = x[:, :, -1].reshape(n)
    y2 = jnp.pad(y0, (0, pad)).reshape(_NR, 128)
    src = edge_index[0].astype(jnp.int32)
    dst = edge_index[1].astype(jnp.int32)

    w0p = _padw(W0, 1, WF)
    w1p = _padw(W1, WF, WF)
    w2p = _padw(W2, WF, WF)
    w3p = _padw(W3, WF, WF)
    w4p = _padw(W4.reshape(64, 1), WF, 1)
    b0p, b1p, b2p, b3p = (
        jnp.pad(b.reshape(1, 64), ((0, 0), (0, WF - 64))) for b in (b0, b1, b2, b3)
    )
    b4r = b4.reshape(1, 1)

    deg = _agg1(jnp.ones((NP,), F32), src, dst)
    dinv2, hs2 = _prep(deg.reshape(NC, _NR, 128), y2)
    dinvcol = dinv2.reshape(NP, 1)

    dt = 1.25
    coefs = [
        (0.0, 0.0, 0.0, dt / 3.0),
        (-dt / 3.0, 0.0, 0.0, dt),
        (dt, -dt, 0.0, dt),
        (dt / 8.0, 3.0 * dt / 8.0, 3.0 * dt / 8.0, dt / 8.0),
    ]
    coefs = [jnp.asarray(c, F32).reshape(1, 4) for c in coefs]

    preds = [y0]
    for _ in range(4):
        ks = []
        for e in range(4):
            u = _agg1(hs2.reshape(NP), src, dst).reshape(NC, NP, 1)
            h1 = _l0(u, hs2.reshape(NP, 1), dinvcol, w0p, b0p)
            u = _aggF(h1, src, dst)
            h2 = _lmid(u, h1, dinvcol, w1p, b1p)
            u = _aggF(h2, src, dst)
            h3 = _lmid(u, h2, dinvcol, w2p, b2p)
            u = _aggF(h3, src, dst)
            zs = _l3z(u, h3, dinvcol, w3p, b3p, w4p)
            u = _agg1(zs.reshape(NP), src, dst).reshape(NC, _NR, 128)
            pads = [y2, y2, y2]
            ka, kb, kc = (ks + pads)[:3]
            knew, ya2, hs2 = _fin(
                u, zs.reshape(_NR, 128), dinv2, b4r, y2, ka, kb, kc, coefs[e]
            )
            ks.append(knew)
        y2 = ya2
        preds.append(y2.reshape(NP)[:n])
    out = jnp.stack([p.reshape(nn) for p in preds], axis=-1)
    return out[None].astype(x.dtype)


# speedup vs baseline: 10.6842x; 1.2620x over previous
import functools

import jax
import jax.numpy as jnp
from jax import lax
from jax.experimental import pallas as pl
from jax.experimental.pallas import tpu as pltpu
from jax.experimental.pallas import tpu_sc as plsc

NN = 10000
NP = 10240
E = 640000
NC, NS = 2, 16
NW = NC * NS
K = 128
NBC = 158
EWP = NBC * K
EP = NW * EWP
SL = NP // NS
ZB = SL // K
WF = 128
F32 = jnp.float32


def _make_edge_agg(width):
    if width == 1:
        out_t = jax.ShapeDtypeStruct((NC, NP), F32)
        rows_t = pltpu.VMEM((2, K), F32)
        acc_t = pltpu.VMEM_SHARED((NP,), F32)
    else:
        out_t = jax.ShapeDtypeStruct((NC, NP, width), F32)
        rows_t = pltpu.VMEM((2, K, width), F32)
        acc_t = pltpu.VMEM_SHARED((NP, width), F32)

    mesh = plsc.VectorSubcoreMesh(
        core_axis_name="c", subcore_axis_name="s", num_cores=NC, num_subcores=NS
    )

    @functools.partial(
        pl.kernel,
        out_type=out_t,
        mesh=mesh,
        scratch_types=[
            pltpu.VMEM((2, K), jnp.int32),
            pltpu.VMEM((2, K), jnp.int32),
            rows_t,
            acc_t,
            pltpu.SemaphoreType.DMA,
            pltpu.SemaphoreType.DMA,
            pltpu.SemaphoreType.DMA,
            pltpu.SemaphoreType.DMA,
            pltpu.SemaphoreType.DMA,
            pltpu.SemaphoreType.DMA,
        ],
    )
    def agg(h_hbm, src_hbm, dst_hbm, out_hbm, sb, db, rows,
            acc, is0, is1, id0, id1, g0, g1):
        c = lax.axis_index("c")
        s = lax.axis_index("s")
        w = c * NS + s
        r0 = rows.at[0]
        r1 = rows.at[1]
        isem = (is0, is1)
        dsem = (id0, id1)
        gsem = (g0, g1)

        def stage(i, p):
            pltpu.async_copy(src_hbm.at[w, i], sb.at[p], isem[p])
            pltpu.async_copy(dst_hbm.at[w, i], db.at[p], dsem[p])

        def wait_idx(i, p):
            pltpu.make_async_copy(src_hbm.at[w, i], sb.at[p], isem[p]).wait()
            pltpu.make_async_copy(dst_hbm.at[w, i], db.at[p], dsem[p]).wait()

        def fire(p, r):
            pltpu.async_copy(h_hbm.at[sb.at[p]], r, gsem[p])

        def wait_g(p, r):
            pltpu.make_async_copy(h_hbm.at[sb.at[p]], r, gsem[p]).wait()

        def scat(p, r):
            if width == 1:
                pltpu.sync_copy(r, acc.at[db.at[p]], add=True)
            else:
                pltpu.sync_copy(r, acc.at[db.at[p]], add=True)

        stage(0, 0)
        stage(1, 1)
        if width == 1:
            r0[...] = jnp.zeros((K,), F32)
            for z in range(ZB):
                pltpu.sync_copy(r0, acc.at[pl.ds(s * SL + z * K, K)])
        else:
            r0[...] = jnp.zeros((K, width), F32)
            for z in range(ZB):
                pltpu.sync_copy(r0, acc.at[pl.ds(s * SL + z * K, K), :])
        wait_idx(0, 0)
        fire(0, r0)
        plsc.subcore_barrier()

        def step(j, carry):
            i = 2 * j
            wait_idx(i + 1, 1)
            fire(1, r1)
            wait_g(0, r0)
            scat(0, r0)
            stage(i + 2, 0)
            wait_g(1, r1)
            scat(1, r1)
            wait_idx(i + 2, 0)
            fire(0, r0)
            stage(i + 3, 1)
            return carry

        lax.fori_loop(0, NBC // 2 - 1, step, 0)
        wait_idx(NBC - 1, 1)
        fire(1, r1)
        wait_g(0, r0)
        scat(0, r0)
        wait_g(1, r1)
        scat(1, r1)

        plsc.subcore_barrier()
        if width == 1:
            pltpu.sync_copy(acc.at[pl.ds(s * SL, SL)], out_hbm.at[c, pl.ds(s * SL, SL)])
        else:
            pltpu.sync_copy(
                acc.at[pl.ds(s * SL, SL), :], out_hbm.at[c, pl.ds(s * SL, SL), :]
            )

    return agg


_agg1 = _make_edge_agg(1)
_aggF = _make_edge_agg(WF)


_R = 2048


_NR = NP // 128


def _prep_body(u_ref, y_ref, dinv_ref, hs_ref):
    indeg = u_ref[0] + u_ref[1]
    dinv = lax.rsqrt(indeg + 1.0)
    dinv_ref[...] = dinv
    hs_ref[...] = dinv * y_ref[...]


def _prep(u, y):
    return pl.pallas_call(
        _prep_body,
        out_shape=(
            jax.ShapeDtypeStruct((_NR, 128), F32),
            jax.ShapeDtypeStruct((_NR, 128), F32),
        ),
    )(u, y)


def _l0_body(u_ref, hs_ref, dinv_ref, w_ref, b_ref, o_ref):
    dinv = dinv_ref[...]
    agg = dinv * (u_ref[0] + u_ref[1] + hs_ref[...])
    h = jnp.tanh(agg * w_ref[...] + b_ref[...])
    o_ref[...] = dinv * h


def _l0(u, hs, dinv, w0, b0):
    g = NP // _R
    return pl.pallas_call(
        _l0_body,
        grid=(g,),
        in_specs=[
            pl.BlockSpec((2, _R, 1), lambda i: (0, i, 0)),
            pl.BlockSpec((_R, 1), lambda i: (i, 0)),
            pl.BlockSpec((_R, 1), lambda i: (i, 0)),
            pl.BlockSpec((1, WF), lambda i: (0, 0)),
            pl.BlockSpec((1, WF), lambda i: (0, 0)),
        ],
        out_specs=pl.BlockSpec((_R, WF), lambda i: (i, 0)),
        out_shape=jax.ShapeDtypeStruct((NP, WF), F32),
    )(u, hs, dinv, w0, b0)


def _lmid_body(u_ref, hs_ref, dinv_ref, w_ref, b_ref, o_ref):
    dinv = dinv_ref[...]
    agg = dinv * (u_ref[0] + u_ref[1] + hs_ref[...])
    h = jnp.tanh(
        jnp.dot(agg, w_ref[...], preferred_element_type=F32) + b_ref[...]
    )
    o_ref[...] = dinv * h


def _lmid(u, hs, dinv, w, b):
    g = NP // _R
    return pl.pallas_call(
        _lmid_body,
        grid=(g,),
        in_specs=[
            pl.BlockSpec((2, _R, WF), lambda i: (0, i, 0)),
            pl.BlockSpec((_R, WF), lambda i: (i, 0)),
            pl.BlockSpec((_R, 1), lambda i: (i, 0)),
            pl.BlockSpec((WF, WF), lambda i: (0, 0)),
            pl.BlockSpec((1, WF), lambda i: (0, 0)),
        ],
        out_specs=pl.BlockSpec((_R, WF), lambda i: (i, 0)),
        out_shape=jax.ShapeDtypeStruct((NP, WF), F32),
    )(u, hs, dinv, w, b)


def _l3z_body(u_ref, hs_ref, dinv_ref, w3_ref, b3_ref, w4_ref, o_ref):
    dinv = dinv_ref[...]
    agg = dinv * (u_ref[0] + u_ref[1] + hs_ref[...])
    h4 = jnp.tanh(
        jnp.dot(agg, w3_ref[...], preferred_element_type=F32) + b3_ref[...]
    )
    z = jnp.dot(h4, w4_ref[...], preferred_element_type=F32)
    o_ref[...] = dinv * z


def _l3z(u, hs, dinv, w3, b3, w4):
    g = NP // _R
    return pl.pallas_call(
        _l3z_body,
        grid=(g,),
        in_specs=[
            pl.BlockSpec((2, _R, WF), lambda i: (0, i, 0)),
            pl.BlockSpec((_R, WF), lambda i: (i, 0)),
            pl.BlockSpec((_R, 1), lambda i: (i, 0)),
            pl.BlockSpec((WF, WF), lambda i: (0, 0)),
            pl.BlockSpec((1, WF), lambda i: (0, 0)),
            pl.BlockSpec((WF, 1), lambda i: (0, 0)),
        ],
        out_specs=pl.BlockSpec((_R, 1), lambda i: (i, 0)),
        out_shape=jax.ShapeDtypeStruct((NP, 1), F32),
    )(u, hs, dinv, w3, b3, w4)


def _fin_body(u_ref, zs_ref, dinv_ref, b4_ref, y_ref, ka_ref, kb_ref, kc_ref,
              coef_ref, k_ref, ya_ref, hsa_ref):
    dinv = dinv_ref[...]
    k = dinv * (u_ref[0] + u_ref[1] + zs_ref[...]) + b4_ref[0, 0]
    y_arg = (
        y_ref[...]
        + coef_ref[0, 0] * ka_ref[...]
        + coef_ref[0, 1] * kb_ref[...]
        + coef_ref[0, 2] * kc_ref[...]
        + coef_ref[0, 3] * k
    )
    k_ref[...] = k
    ya_ref[...] = y_arg
    hsa_ref[...] = dinv * y_arg


def _fin(u, zs, dinv, b4, y, ka, kb, kc, coef):
    return pl.pallas_call(
        _fin_body,
        out_shape=(
            jax.ShapeDtypeStruct((_NR, 128), F32),
            jax.ShapeDtypeStruct((_NR, 128), F32),
            jax.ShapeDtypeStruct((_NR, 128), F32),
        ),
    )(u, zs, dinv, b4, y, ka, kb, kc, coef)


def _padw(w, rows, cols):
    return jnp.pad(w, ((0, rows - w.shape[0]), (0, cols - w.shape[1])))


def kernel(x, edge_index, W0, b0, W1, b1, W2, b2, W3, b3, W4, b4):
    bsz, nn, _ = x.shape
    n = bsz * nn
    pad = NP - n
    y0 = x[:, :, -1].reshape(n)
    y2 = jnp.pad(y0, (0, pad)).reshape(_NR, 128)
    epad = jnp.full((EP - E,), NP - 1, jnp.int32)
    src = jnp.concatenate([edge_index[0].astype(jnp.int32), epad]).reshape(NW, NBC, K)
    dst = jnp.concatenate([edge_index[1].astype(jnp.int32), epad]).reshape(NW, NBC, K)

    w0p = _padw(W0, 1, WF)
    w1p = _padw(W1, WF, WF)
    w2p = _padw(W2, WF, WF)
    w3p = _padw(W3, WF, WF)
    w4p = _padw(W4.reshape(64, 1), WF, 1)
    b0p, b1p, b2p, b3p = (
        jnp.pad(b.reshape(1, 64), ((0, 0), (0, WF - 64))) for b in (b0, b1, b2, b3)
    )
    b4r = b4.reshape(1, 1)

    deg = _agg1(jnp.ones((NP,), F32), src, dst)
    dinv2, hs2 = _prep(deg.reshape(NC, _NR, 128), y2)
    dinvcol = dinv2.reshape(NP, 1)

    dt = 1.25
    coefs = [
        (0.0, 0.0, 0.0, dt / 3.0),
        (-dt / 3.0, 0.0, 0.0, dt),
        (dt, -dt, 0.0, dt),
        (dt / 8.0, 3.0 * dt / 8.0, 3.0 * dt / 8.0, dt / 8.0),
    ]
    coefs = [jnp.asarray(c, F32).reshape(1, 4) for c in coefs]

    preds = [y0]
    for _ in range(4):
        ks = []
        for e in range(4):
            u = _agg1(hs2.reshape(NP), src, dst).reshape(NC, NP, 1)
            h1 = _l0(u, hs2.reshape(NP, 1), dinvcol, w0p, b0p)
            u = _aggF(h1, src, dst)
            h2 = _lmid(u, h1, dinvcol, w1p, b1p)
            u = _aggF(h2, src, dst)
            h3 = _lmid(u, h2, dinvcol, w2p, b2p)
            u = _aggF(h3, src, dst)
            zs = _l3z(u, h3, dinvcol, w3p, b3p, w4p)
            u = _agg1(zs.reshape(NP), src, dst).reshape(NC, _NR, 128)
            pads = [y2, y2, y2]
            ka, kb, kc = (ks + pads)[:3]
            knew, ya2, hs2 = _fin(
                u, zs.reshape(_NR, 128), dinv2, b4r, y2, ka, kb, kc, coefs[e]
            )
            ks.append(knew)
        y2 = ya2
        preds.append(y2.reshape(NP)[:n])
    out = jnp.stack([p.reshape(nn) for p in preds], axis=-1)
    return out[None].astype(x.dtype)
